# baseline (speedup 1.0000x reference)
import functools

import jax
import jax.numpy as jnp
from jax import lax
from jax.experimental import pallas as pl
from jax.experimental.pallas import tpu as pltpu
from jax.experimental.pallas import tpu_sc as plsc

N_NODES = 10000
N_EDGES = 320000
D_FEAT = 128
NUM_GRAPHS = 64

_NC = 2
_NS = 16
_NW = _NC * _NS
_CHUNK = 128
_NCHUNK = 80
_EDGES_PAD = _NW * _NCHUNK * _CHUNK
_NBUF = 2
_GRP = 20
_NGRP = _NCHUNK // _GRP
_ACC_ROWS = N_NODES + _CHUNK
_ROWS_PER_TILE = 624
_ROWS_TAIL = _ACC_ROWS - _NS * _ROWS_PER_TILE


def _segsum_body(h_hbm, ei_hbm, zeros_hbm, out_hbm,
                 iblk0, iblk1, rows0, rows1, acc, gs0, gs1, is0, is1):
    cid = lax.axis_index("c")
    sid = lax.axis_index("s")
    wid = sid * _NC + cid
    iblks = [iblk0, iblk1]
    rows = [rows0, rows1]
    gsem = [gs0, gs1]
    isem = [is0, is1]

    zd = pltpu.async_copy(
        zeros_hbm, acc.at[pl.ds(sid * _ROWS_PER_TILE, _ROWS_PER_TILE)], isem[0])

    @pl.when(sid == _NS - 1)
    def _():
        pltpu.sync_copy(zeros_hbm.at[pl.ds(0, _ROWS_TAIL)],
                        acc.at[pl.ds(_NS * _ROWS_PER_TILE, _ROWS_TAIL)])

    pltpu.sync_copy(ei_hbm.at[wid, 0], iblk0)
    pltpu.async_copy(ei_hbm.at[wid, 1], iblk1, isem[1])
    pltpu.async_copy(h_hbm.at[iblk0.at[0, 0]], rows0, gsem[0])
    pltpu.async_copy(h_hbm.at[iblk0.at[0, 1]], rows1, gsem[1])
    zd.wait()
    plsc.subcore_barrier()

    def body(gg, carry):
        for gi in range(2):
            g = gg * 2 + gi
            x = iblks[gi]
            y = iblks[1 - gi]
            for k in range(_GRP):
                t = g * _GRP + k
                s = k % 2
                pltpu.make_async_copy(h_hbm.at[x.at[0, k]], rows[s],
                                      gsem[s]).wait()
                pltpu.sync_copy(rows[s], acc.at[x.at[1, k]], add=True)
                if k < _GRP - 2:
                    pltpu.async_copy(h_hbm.at[x.at[0, k + 2]], rows[s], gsem[s])
                else:
                    if k == _GRP - 2:
                        @pl.when(g + 1 < _NGRP)
                        def _():
                            pltpu.make_async_copy(ei_hbm.at[wid, g + 1], y,
                                                  isem[1 - gi]).wait()

                    @pl.when(t + 2 < _NCHUNK)
                    def _():
                        pltpu.async_copy(h_hbm.at[y.at[0, k - (_GRP - 2)]],
                                         rows[s], gsem[s])

            @pl.when(g + 2 < _NGRP)
            def _():
                pltpu.async_copy(ei_hbm.at[wid, g + 2], x, isem[gi])
        return carry

    lax.fori_loop(0, _NGRP // 2, body, 0)

    plsc.subcore_barrier()

    r0 = sid * _ROWS_PER_TILE
    pltpu.sync_copy(acc.at[pl.ds(r0, _ROWS_PER_TILE)],
                    out_hbm.at[cid, pl.ds(r0, _ROWS_PER_TILE)])

    @pl.when(sid == _NS - 1)
    def _():
        t0 = _NS * _ROWS_PER_TILE
        pltpu.sync_copy(acc.at[pl.ds(t0, N_NODES - t0)],
                        out_hbm.at[cid, pl.ds(t0, N_NODES - t0)])


_segsum = pl.kernel(
    _segsum_body,
    out_type=jax.ShapeDtypeStruct((_NC, N_NODES, D_FEAT), jnp.float32),
    mesh=plsc.VectorSubcoreMesh(core_axis_name="c", subcore_axis_name="s"),
    scratch_types=[
        pltpu.VMEM((2, _GRP, _CHUNK), jnp.int32),
        pltpu.VMEM((2, _GRP, _CHUNK), jnp.int32),
        pltpu.VMEM((_CHUNK, D_FEAT), jnp.float32),
        pltpu.VMEM((_CHUNK, D_FEAT), jnp.float32),
        pltpu.VMEM_SHARED((_ACC_ROWS, D_FEAT), jnp.float32),
        pltpu.SemaphoreType.DMA,
        pltpu.SemaphoreType.DMA,
        pltpu.SemaphoreType.DMA,
        pltpu.SemaphoreType.DMA,
    ],
)


def _mlp_body(h_ref, agg_ref, batch_ref, w1_ref, b1_ref, g_ref, be_ref,
              w2_ref, b2_ref, hout_ref, pooled_ref):
    h = h_ref[...]
    out = h + agg_ref[0] + agg_ref[1]
    out = jnp.dot(out, w1_ref[...], preferred_element_type=jnp.float32) + b1_ref[...]
    mean = jnp.mean(out, axis=0, keepdims=True)
    var = jnp.mean(jnp.square(out - mean), axis=0, keepdims=True)
    out = (out - mean) * lax.rsqrt(var + 1e-5) * g_ref[...] + be_ref[...]
    out = jnp.maximum(out, 0.0)
    out = jnp.dot(out, w2_ref[...], preferred_element_type=jnp.float32) + b2_ref[...]
    h2 = jnp.maximum(out, 0.0)
    hout_ref[...] = h2

    gids = lax.broadcasted_iota(jnp.int32, (N_NODES, NUM_GRAPHS), 1)
    mask = (batch_ref[...] == gids).astype(jnp.float32)
    sums = lax.dot_general(mask, h2, (((0,), (0,)), ((), ())),
                           preferred_element_type=jnp.float32)
    counts = jnp.sum(mask, axis=0)[:, None]
    pooled_ref[...] = sums / jnp.maximum(counts, 1.0)


_mlp = pl.pallas_call(
    _mlp_body,
    out_shape=(
        jax.ShapeDtypeStruct((N_NODES, D_FEAT), jnp.float32),
        jax.ShapeDtypeStruct((NUM_GRAPHS, D_FEAT), jnp.float32),
    ),
)


def _mlp3_body(h_ref, agg_ref, batch_ref, w1_ref, b1_ref, g_ref, be_ref,
               w2_ref, b2_ref, p0_ref, p1_ref, wp_ref, bp_ref, wc_ref, bc_ref,
               z_ref, logits_ref):
    h = h_ref[...]
    out = h + agg_ref[0] + agg_ref[1]
    out = jnp.dot(out, w1_ref[...], preferred_element_type=jnp.float32) + b1_ref[...]
    mean = jnp.mean(out, axis=0, keepdims=True)
    var = jnp.mean(jnp.square(out - mean), axis=0, keepdims=True)
    out = (out - mean) * lax.rsqrt(var + 1e-5) * g_ref[...] + be_ref[...]
    out = jnp.maximum(out, 0.0)
    out = jnp.dot(out, w2_ref[...], preferred_element_type=jnp.float32) + b2_ref[...]
    h2 = jnp.maximum(out, 0.0)

    gids = lax.broadcasted_iota(jnp.int32, (N_NODES, NUM_GRAPHS), 1)
    mask = (batch_ref[...] == gids).astype(jnp.float32)
    sums = lax.dot_general(mask, h2, (((0,), (0,)), ((), ())),
                           preferred_element_type=jnp.float32)
    counts = jnp.sum(mask, axis=0)[:, None]
    p2 = sums / jnp.maximum(counts, 1.0)

    hjk = jnp.concatenate([p0_ref[...], p1_ref[...], p2], axis=1)
    z = jnp.dot(hjk, wp_ref[...], preferred_element_type=jnp.float32) + bp_ref[...]
    z_ref[...] = z
    logits_ref[...] = (
        jnp.dot(z, wc_ref[...], preferred_element_type=jnp.float32) + bc_ref[...])


def kernel(x, edge_index, batch, params):
    pad = _EDGES_PAD - N_EDGES
    srcp = jnp.concatenate(
        [edge_index[0], jnp.arange(pad, dtype=jnp.int32) % _CHUNK]
    ).reshape(_NW, _NGRP, _GRP, _CHUNK)
    dstp = jnp.concatenate(
        [edge_index[1], N_NODES + (jnp.arange(pad, dtype=jnp.int32) % _CHUNK)]
    ).reshape(_NW, _NGRP, _GRP, _CHUNK)
    ei = jnp.stack([srcp, dstp], axis=2)
    batch2 = batch[:, None]
    zeros = jnp.zeros((_ROWS_PER_TILE, D_FEAT), jnp.float32)

    h = x
    pooled = []
    for i in range(2):
        p = params['conv%d' % i]
        agg = _segsum(h, ei, zeros)
        h, pool = _mlp(h, agg, batch2,
                       p['W1'], p['b1'][None, :], p['gamma'][None, :],
                       p['beta'][None, :], p['W2'], p['b2'][None, :])
        pooled.append(pool)

    p = params['conv2']
    agg = _segsum(h, ei, zeros)
    mlp3 = pl.pallas_call(
        _mlp3_body,
        out_shape=(
            jax.ShapeDtypeStruct((NUM_GRAPHS, params['Wp'].shape[1]), jnp.float32),
            jax.ShapeDtypeStruct((NUM_GRAPHS, params['Wc'].shape[1]), jnp.float32),
        ),
    )
    z, logits = mlp3(h, agg, batch2,
                     p['W1'], p['b1'][None, :], p['gamma'][None, :],
                     p['beta'][None, :], p['W2'], p['b2'][None, :],
                     pooled[0], pooled[1],
                     params['Wp'], params['bp'][None, :],
                     params['Wc'], params['bc'][None, :])
    return z, logits


# speedup vs baseline: 1.0047x; 1.0047x over previous
import functools

import jax
import jax.numpy as jnp
from jax import lax
from jax.experimental import pallas as pl
from jax.experimental.pallas import tpu as pltpu
from jax.experimental.pallas import tpu_sc as plsc

N_NODES = 10000
N_EDGES = 320000
D_FEAT = 128
NUM_GRAPHS = 64

_NC = 2
_NS = 16
_NW = _NC * _NS
_CHUNK = 128
_NCHUNK = 80
_EDGES_PAD = _NW * _NCHUNK * _CHUNK
_NBUF = 2
_GRP = 10
_NGRP = _NCHUNK // _GRP
_ACC_ROWS = N_NODES + _CHUNK
_ROWS_PER_TILE = 624
_ROWS_TAIL = _ACC_ROWS - _NS * _ROWS_PER_TILE


def _segsum_body(h_hbm, ei_hbm, zeros_hbm, out_hbm,
                 iblk0, iblk1, rows0, rows1, acc, gs0, gs1, is0, is1):
    cid = lax.axis_index("c")
    sid = lax.axis_index("s")
    wid = sid * _NC + cid
    iblks = [iblk0, iblk1]
    rows = [rows0, rows1]
    gsem = [gs0, gs1]
    isem = [is0, is1]

    zd = pltpu.async_copy(
        zeros_hbm, acc.at[pl.ds(sid * _ROWS_PER_TILE, _ROWS_PER_TILE)], isem[0])

    @pl.when(sid == _NS - 1)
    def _():
        pltpu.sync_copy(zeros_hbm.at[pl.ds(0, _ROWS_TAIL)],
                        acc.at[pl.ds(_NS * _ROWS_PER_TILE, _ROWS_TAIL)])

    pltpu.sync_copy(ei_hbm.at[wid, 0], iblk0)
    pltpu.async_copy(ei_hbm.at[wid, 1], iblk1, isem[1])
    pltpu.async_copy(h_hbm.at[iblk0.at[0, 0]], rows0, gsem[0])
    pltpu.async_copy(h_hbm.at[iblk0.at[0, 1]], rows1, gsem[1])
    zd.wait()
    plsc.subcore_barrier()

    def body(gg, carry):
        for gi in range(2):
            g = gg * 2 + gi
            x = iblks[gi]
            y = iblks[1 - gi]
            for k in range(_GRP):
                t = g * _GRP + k
                s = k % 2
                pltpu.make_async_copy(h_hbm.at[x.at[0, k]], rows[s],
                                      gsem[s]).wait()
                pltpu.sync_copy(rows[s], acc.at[x.at[1, k]], add=True)
                if k < _GRP - 2:
                    pltpu.async_copy(h_hbm.at[x.at[0, k + 2]], rows[s], gsem[s])
                else:
                    if k == _GRP - 2:
                        @pl.when(g + 1 < _NGRP)
                        def _():
                            pltpu.make_async_copy(ei_hbm.at[wid, g + 1], y,
                                                  isem[1 - gi]).wait()

                    @pl.when(t + 2 < _NCHUNK)
                    def _():
                        pltpu.async_copy(h_hbm.at[y.at[0, k - (_GRP - 2)]],
                                         rows[s], gsem[s])

            @pl.when(g + 2 < _NGRP)
            def _():
                pltpu.async_copy(ei_hbm.at[wid, g + 2], x, isem[gi])
        return carry

    lax.fori_loop(0, _NGRP // 2, body, 0)

    plsc.subcore_barrier()

    r0 = sid * _ROWS_PER_TILE
    pltpu.sync_copy(acc.at[pl.ds(r0, _ROWS_PER_TILE)],
                    out_hbm.at[cid, pl.ds(r0, _ROWS_PER_TILE)])

    @pl.when(sid == _NS - 1)
    def _():
        t0 = _NS * _ROWS_PER_TILE
        pltpu.sync_copy(acc.at[pl.ds(t0, N_NODES - t0)],
                        out_hbm.at[cid, pl.ds(t0, N_NODES - t0)])


_segsum = pl.kernel(
    _segsum_body,
    out_type=jax.ShapeDtypeStruct((_NC, N_NODES, D_FEAT), jnp.float32),
    mesh=plsc.VectorSubcoreMesh(core_axis_name="c", subcore_axis_name="s"),
    scratch_types=[
        pltpu.VMEM((2, _GRP, _CHUNK), jnp.int32),
        pltpu.VMEM((2, _GRP, _CHUNK), jnp.int32),
        pltpu.VMEM((_CHUNK, D_FEAT), jnp.float32),
        pltpu.VMEM((_CHUNK, D_FEAT), jnp.float32),
        pltpu.VMEM_SHARED((_ACC_ROWS, D_FEAT), jnp.float32),
        pltpu.SemaphoreType.DMA,
        pltpu.SemaphoreType.DMA,
        pltpu.SemaphoreType.DMA,
        pltpu.SemaphoreType.DMA,
    ],
)


def _mlp_body(h_ref, agg_ref, batch_ref, w1_ref, b1_ref, g_ref, be_ref,
              w2_ref, b2_ref, hout_ref, pooled_ref):
    h = h_ref[...]
    out = h + agg_ref[0] + agg_ref[1]
    out = jnp.dot(out, w1_ref[...], preferred_element_type=jnp.float32) + b1_ref[...]
    mean = jnp.mean(out, axis=0, keepdims=True)
    var = jnp.mean(jnp.square(out - mean), axis=0, keepdims=True)
    out = (out - mean) * lax.rsqrt(var + 1e-5) * g_ref[...] + be_ref[...]
    out = jnp.maximum(out, 0.0)
    out = jnp.dot(out, w2_ref[...], preferred_element_type=jnp.float32) + b2_ref[...]
    h2 = jnp.maximum(out, 0.0)
    hout_ref[...] = h2

    gids = lax.broadcasted_iota(jnp.int32, (N_NODES, NUM_GRAPHS), 1)
    mask = (batch_ref[...] == gids).astype(jnp.float32)
    sums = lax.dot_general(mask, h2, (((0,), (0,)), ((), ())),
                           preferred_element_type=jnp.float32)
    counts = jnp.sum(mask, axis=0)[:, None]
    pooled_ref[...] = sums / jnp.maximum(counts, 1.0)


_mlp = pl.pallas_call(
    _mlp_body,
    out_shape=(
        jax.ShapeDtypeStruct((N_NODES, D_FEAT), jnp.float32),
        jax.ShapeDtypeStruct((NUM_GRAPHS, D_FEAT), jnp.float32),
    ),
)


def _mlp3_body(h_ref, agg_ref, batch_ref, w1_ref, b1_ref, g_ref, be_ref,
               w2_ref, b2_ref, p0_ref, p1_ref, wp_ref, bp_ref, wc_ref, bc_ref,
               z_ref, logits_ref):
    h = h_ref[...]
    out = h + agg_ref[0] + agg_ref[1]
    out = jnp.dot(out, w1_ref[...], preferred_element_type=jnp.float32) + b1_ref[...]
    mean = jnp.mean(out, axis=0, keepdims=True)
    var = jnp.mean(jnp.square(out - mean), axis=0, keepdims=True)
    out = (out - mean) * lax.rsqrt(var + 1e-5) * g_ref[...] + be_ref[...]
    out = jnp.maximum(out, 0.0)
    out = jnp.dot(out, w2_ref[...], preferred_element_type=jnp.float32) + b2_ref[...]
    h2 = jnp.maximum(out, 0.0)

    gids = lax.broadcasted_iota(jnp.int32, (N_NODES, NUM_GRAPHS), 1)
    mask = (batch_ref[...] == gids).astype(jnp.float32)
    sums = lax.dot_general(mask, h2, (((0,), (0,)), ((), ())),
                           preferred_element_type=jnp.float32)
    counts = jnp.sum(mask, axis=0)[:, None]
    p2 = sums / jnp.maximum(counts, 1.0)

    hjk = jnp.concatenate([p0_ref[...], p1_ref[...], p2], axis=1)
    z = jnp.dot(hjk, wp_ref[...], preferred_element_type=jnp.float32) + bp_ref[...]
    z_ref[...] = z
    logits_ref[...] = (
        jnp.dot(z, wc_ref[...], preferred_element_type=jnp.float32) + bc_ref[...])


def kernel(x, edge_index, batch, params):
    pad = _EDGES_PAD - N_EDGES
    srcp = jnp.concatenate(
        [edge_index[0], jnp.arange(pad, dtype=jnp.int32) % _CHUNK]
    ).reshape(_NW, _NGRP, _GRP, _CHUNK)
    dstp = jnp.concatenate(
        [edge_index[1], N_NODES + (jnp.arange(pad, dtype=jnp.int32) % _CHUNK)]
    ).reshape(_NW, _NGRP, _GRP, _CHUNK)
    ei = jnp.stack([srcp, dstp], axis=2)
    batch2 = batch[:, None]
    zeros = jnp.zeros((_ROWS_PER_TILE, D_FEAT), jnp.float32)

    h = x
    pooled = []
    for i in range(2):
        p = params['conv%d' % i]
        agg = _segsum(h, ei, zeros)
        h, pool = _mlp(h, agg, batch2,
                       p['W1'], p['b1'][None, :], p['gamma'][None, :],
                       p['beta'][None, :], p['W2'], p['b2'][None, :])
        pooled.append(pool)

    p = params['conv2']
    agg = _segsum(h, ei, zeros)
    mlp3 = pl.pallas_call(
        _mlp3_body,
        out_shape=(
            jax.ShapeDtypeStruct((NUM_GRAPHS, params['Wp'].shape[1]), jnp.float32),
            jax.ShapeDtypeStruct((NUM_GRAPHS, params['Wc'].shape[1]), jnp.float32),
        ),
    )
    z, logits = mlp3(h, agg, batch2,
                     p['W1'], p['b1'][None, :], p['gamma'][None, :],
                     p['beta'][None, :], p['W2'], p['b2'][None, :],
                     pooled[0], pooled[1],
                     params['Wp'], params['bp'][None, :],
                     params['Wc'], params['bc'][None, :])
    return z, logits
